# Initial kernel scaffold; baseline (speedup 1.0000x reference)
import jax
import jax.numpy as jnp
from jax.experimental import pallas as pl


def kernel(X, edge_index, edge_weight, W_z, b_z, W_r, b_r, W_h, b_h):
    raise NotImplementedError("write your pallas kernel here")


# speedup vs baseline: 10.4265x; 10.4265x over previous
import functools

import jax
import jax.numpy as jnp
from jax import lax
from jax.experimental import pallas as pl
from jax.experimental.pallas import tpu as pltpu
from jax.experimental.pallas import tpu_sc as plsc

N = 10000
D = 128
HALF = 64
E = 320000
NC = 2
NS = 16
LANES = 16
N_PAD = 10240
RPS = N_PAD // NS
CHUNK = 80

_MESH = plsc.VectorSubcoreMesh(core_axis_name="core", subcore_axis_name="subcore")

_F32 = jnp.float32


@functools.partial(
    pl.kernel,
    out_type=jax.ShapeDtypeStruct((NC * 2 * N_PAD, LANES), _F32),
    mesh=_MESH,
    scratch_types=[
        pltpu.VMEM((CHUNK,), _F32),
        pltpu.VMEM((CHUNK,), jnp.int32),
        pltpu.VMEM((CHUNK,), jnp.int32),
        pltpu.VMEM((CHUNK, LANES), _F32),
        pltpu.VMEM_SHARED((N_PAD, LANES), _F32),
        pltpu.VMEM_SHARED((N_PAD, LANES), _F32),
    ],
)
def _deg_kernel(w_hbm, row_hbm, col_hbm, z16_hbm, out_hbm,
                wv, rv, cv, mv, acc_o, acc_i):
    c = lax.axis_index("core")
    s = lax.axis_index("subcore")
    wid = c * NS + s
    iota = lax.broadcasted_iota(jnp.int32, (LANES,), 0)

    pltpu.sync_copy(z16_hbm, acc_o.at[pl.ds(s * RPS, RPS)])
    pltpu.sync_copy(z16_hbm, acc_i.at[pl.ds(s * RPS, RPS)])
    plsc.subcore_barrier()

    epw = E // (NC * NS)
    base_w = wid * epw

    @pl.loop(0, epw // CHUNK)
    def _(i):
        base = base_w + i * CHUNK
        pltpu.sync_copy(w_hbm.at[pl.ds(base, CHUNK)], wv)
        pltpu.sync_copy(row_hbm.at[pl.ds(base, CHUNK)], rv)
        pltpu.sync_copy(col_hbm.at[pl.ds(base, CHUNK)], cv)
        for g in range(CHUNK // LANES):
            w16 = wv[pl.ds(g * LANES, LANES)]
            for b in range(LANES):
                mv[g * LANES + b, :] = jnp.where(iota == b, w16, 0.0)
        pltpu.sync_copy(mv, acc_o.at[rv], add=True)
        pltpu.sync_copy(mv, acc_i.at[cv], add=True)

    plsc.subcore_barrier()
    pltpu.sync_copy(acc_o.at[pl.ds(s * RPS, RPS)],
                    out_hbm.at[pl.ds((c * 2) * N_PAD + s * RPS, RPS)])
    pltpu.sync_copy(acc_i.at[pl.ds(s * RPS, RPS)],
                    out_hbm.at[pl.ds((c * 2 + 1) * N_PAD + s * RPS, RPS)])


@functools.partial(
    pl.kernel,
    out_type=jax.ShapeDtypeStruct((NC, N_PAD, D), _F32),
    mesh=_MESH,
    scratch_types=[
        pltpu.VMEM((CHUNK,), jnp.int32),
        pltpu.VMEM((CHUNK,), jnp.int32),
        pltpu.VMEM((CHUNK, D), _F32),
        pltpu.VMEM_SHARED((N_PAD, D), _F32),
    ],
)
def _hop_kernel(y_hbm, rowsab_hbm, col_hbm, z128_hbm, t_hbm,
                rv, cv, gb, acc):
    c = lax.axis_index("core")
    s = lax.axis_index("subcore")

    pltpu.sync_copy(z128_hbm, acc.at[pl.ds(s * RPS, RPS)])
    plsc.subcore_barrier()

    epw = E // NS
    base_s = s * epw

    @pl.loop(0, epw // CHUNK)
    def _(i):
        base = base_s + i * CHUNK
        pltpu.sync_copy(rowsab_hbm.at[pl.ds(c * E + base, CHUNK)], rv)
        pltpu.sync_copy(col_hbm.at[pl.ds(base, CHUNK)], cv)
        pltpu.sync_copy(y_hbm.at[rv], gb)
        pltpu.sync_copy(gb, acc.at[cv], add=True)

    plsc.subcore_barrier()
    pltpu.sync_copy(acc.at[pl.ds(s * RPS, RPS)], t_hbm.at[c, pl.ds(s * RPS, RPS)])


_BN = 1000


def _scale1_body(ds_ref, x_ref, yc_ref, dvo_ref, dvi_ref):
    ds = ds_ref[...]
    deg_o = jnp.sum(ds[:, 0], axis=(0, 2))
    deg_i = jnp.sum(ds[:, 1], axis=(0, 2))
    dinv_o = 1.0 / deg_o
    dinv_i = 1.0 / deg_i
    x = x_ref[...]
    yo = x * dinv_o[:, None]
    yi = x * dinv_i[:, None]
    yc_ref[0] = jnp.concatenate([yo[:, :HALF], yi[:, :HALF]], axis=1)
    yc_ref[1] = jnp.concatenate([yo[:, HALF:], yi[:, HALF:]], axis=1)
    dvo_ref[...] = dinv_o[:, None]
    dvi_ref[...] = dinv_i[:, None]


_scale1 = pl.pallas_call(
    _scale1_body,
    grid=(N // _BN,),
    in_specs=[
        pl.BlockSpec((NC, 2, _BN, LANES), lambda i: (0, 0, i, 0)),
        pl.BlockSpec((_BN, D), lambda i: (i, 0)),
    ],
    out_specs=[
        pl.BlockSpec((NC, _BN, D), lambda i: (0, i, 0)),
        pl.BlockSpec((_BN, 1), lambda i: (i, 0)),
        pl.BlockSpec((_BN, 1), lambda i: (i, 0)),
    ],
    out_shape=[
        jax.ShapeDtypeStruct((NC, N, D), _F32),
        jax.ShapeDtypeStruct((N, 1), _F32),
        jax.ShapeDtypeStruct((N, 1), _F32),
    ],
)


def _scale2_body(t_ref, dvo_ref, dvi_ref, yc_ref):
    dinv_o = dvo_ref[...]
    dinv_i = dvi_ref[...]
    for cc in range(NC):
        t = t_ref[cc]
        yc_ref[cc] = jnp.concatenate(
            [t[:, :HALF] * dinv_o, t[:, HALF:] * dinv_i], axis=1)


_scale2 = pl.pallas_call(
    _scale2_body,
    grid=(N // _BN,),
    in_specs=[
        pl.BlockSpec((NC, _BN, D), lambda i: (0, i, 0)),
        pl.BlockSpec((_BN, 1), lambda i: (i, 0)),
        pl.BlockSpec((_BN, 1), lambda i: (i, 0)),
    ],
    out_specs=pl.BlockSpec((NC, _BN, D), lambda i: (0, i, 0)),
    out_shape=jax.ShapeDtypeStruct((NC, N, D), _F32),
)


def _final_body(x_ref, t1_ref, p2_ref, wx_ref, wh_ref, b_ref, out_ref):
    dn = (((1,), (0,)), ((), ()))
    hi = lax.Precision.HIGHEST
    o = lax.dot_general(x_ref[...], wx_ref[...], dn, precision=hi)
    for a, ref in enumerate((t1_ref, p2_ref)):
        for cc in range(NC):
            t = ref[cc]
            o += lax.dot_general(t[:, :HALF], wh_ref[a, cc, 0], dn, precision=hi)
            o += lax.dot_general(t[:, HALF:], wh_ref[a, cc, 1], dn, precision=hi)
    o += b_ref[...]
    z = jax.nn.sigmoid(o[:, :D])
    ht = jnp.tanh(o[:, D:])
    out_ref[...] = (1.0 - z) * ht


_final = pl.pallas_call(
    _final_body,
    grid=(N // _BN,),
    in_specs=[
        pl.BlockSpec((_BN, D), lambda i: (i, 0)),
        pl.BlockSpec((NC, _BN, D), lambda i: (0, i, 0)),
        pl.BlockSpec((NC, _BN, D), lambda i: (0, i, 0)),
        pl.BlockSpec((D, 2 * D), lambda i: (0, 0)),
        pl.BlockSpec((2, NC, 2, HALF, 2 * D), lambda i: (0, 0, 0, 0, 0)),
        pl.BlockSpec((1, 2 * D), lambda i: (0, 0)),
    ],
    out_specs=pl.BlockSpec((_BN, D), lambda i: (i, 0)),
    out_shape=jax.ShapeDtypeStruct((N, D), _F32),
)


def kernel(X, edge_index, edge_weight, W_z, b_z, W_r, b_r, W_h, b_h):
    row = edge_index[0].astype(jnp.int32)
    col = edge_index[1].astype(jnp.int32)
    w = edge_weight.astype(_F32)
    rowsab = jnp.concatenate([row, row + N])
    z16 = jnp.zeros((RPS, LANES), _F32)
    z128 = jnp.zeros((RPS, D), _F32)

    ds = _deg_kernel(w, row, col, z16).reshape(NC, 2, N_PAD, LANES)
    y1, dvo, dvi = _scale1(ds, X)
    t1 = _hop_kernel(y1.reshape(NC * N, D), rowsab, col, z128)
    y2 = _scale2(t1, dvo, dvi)
    p2 = _hop_kernel(y2.reshape(NC * N, D), rowsab, col, z128)

    def eff(Wg):
        Wk = Wg[:, :, :D, :]
        wa = Wk[0, 0] + Wk[1, 0] - Wk[0, 2] - Wk[1, 2]
        return wa, Wk[0, 1], Wk[1, 1], 2.0 * Wk[0, 2], 2.0 * Wk[1, 2]

    az = eff(W_z)
    ah = eff(W_h)
    wx = jnp.concatenate([az[0], ah[0]], axis=1)
    wh_rows = []
    for a in range(2):
        core_rows = []
        for cc in range(NC):
            kind_rows = []
            for k in range(2):
                wcat = jnp.concatenate(
                    [az[1 + 2 * a + k], ah[1 + 2 * a + k]], axis=1)
                kind_rows.append(wcat[cc * HALF:(cc + 1) * HALF])
            core_rows.append(jnp.stack(kind_rows))
        wh_rows.append(jnp.stack(core_rows))
    wh = jnp.stack(wh_rows)
    b2 = jnp.concatenate([b_z, b_h]).reshape(1, 2 * D)

    return _final(X, t1, p2, wx, wh, b2)


# speedup vs baseline: 20.0131x; 1.9194x over previous
import functools

import jax
import jax.numpy as jnp
from jax import lax
from jax.experimental import pallas as pl
from jax.experimental.pallas import tpu as pltpu
from jax.experimental.pallas import tpu_sc as plsc

N = 10000
D = 128
HALF = 64
E = 320000
NC = 2
NS = 16
LANES = 16
N_PAD = 10240
RPS = N_PAD // NS
CHUNK = 80

_MESH = plsc.VectorSubcoreMesh(core_axis_name="core", subcore_axis_name="subcore")

_F32 = jnp.float32


@functools.partial(
    pl.kernel,
    out_type=jax.ShapeDtypeStruct((NC * 2 * N_PAD, LANES), _F32),
    mesh=_MESH,
    scratch_types=[
        pltpu.VMEM((CHUNK,), _F32),
        pltpu.VMEM((CHUNK,), jnp.int32),
        pltpu.VMEM((CHUNK,), jnp.int32),
        pltpu.VMEM((CHUNK, LANES), _F32),
        pltpu.VMEM_SHARED((N_PAD, LANES), _F32),
        pltpu.VMEM_SHARED((N_PAD, LANES), _F32),
    ],
)
def _deg_kernel(w_hbm, row_hbm, col_hbm, z16_hbm, out_hbm,
                wv, rv, cv, mv, acc_o, acc_i):
    c = lax.axis_index("core")
    s = lax.axis_index("subcore")
    wid = c * NS + s
    iota = lax.broadcasted_iota(jnp.int32, (LANES,), 0)

    pltpu.sync_copy(z16_hbm, acc_o.at[pl.ds(s * RPS, RPS)])
    pltpu.sync_copy(z16_hbm, acc_i.at[pl.ds(s * RPS, RPS)])
    plsc.subcore_barrier()

    epw = E // (NC * NS)
    base_w = wid * epw

    @pl.loop(0, epw // CHUNK)
    def _(i):
        base = base_w + i * CHUNK
        pltpu.sync_copy(w_hbm.at[pl.ds(base, CHUNK)], wv)
        pltpu.sync_copy(row_hbm.at[pl.ds(base, CHUNK)], rv)
        pltpu.sync_copy(col_hbm.at[pl.ds(base, CHUNK)], cv)
        for g in range(CHUNK // LANES):
            w16 = wv[pl.ds(g * LANES, LANES)]
            for b in range(LANES):
                mv[g * LANES + b, :] = jnp.where(iota == b, w16, 0.0)
        pltpu.sync_copy(mv, acc_o.at[rv], add=True)
        pltpu.sync_copy(mv, acc_i.at[cv], add=True)

    plsc.subcore_barrier()
    pltpu.sync_copy(acc_o.at[pl.ds(s * RPS, RPS)],
                    out_hbm.at[pl.ds((c * 2) * N_PAD + s * RPS, RPS)])
    pltpu.sync_copy(acc_i.at[pl.ds(s * RPS, RPS)],
                    out_hbm.at[pl.ds((c * 2 + 1) * N_PAD + s * RPS, RPS)])


NCH = E // NS // CHUNK
NCHC = E // CHUNK
SUPER = 50
NSUP = NCH // SUPER


@functools.partial(
    pl.kernel,
    out_type=jax.ShapeDtypeStruct((NC, N_PAD, D), _F32),
    mesh=_MESH,
    scratch_types=[
        pltpu.VMEM((SUPER, 2, CHUNK), jnp.int32),
        pltpu.VMEM((CHUNK, D), _F32),
        pltpu.VMEM((CHUNK, D), _F32),
        pltpu.VMEM_SHARED((N_PAD, D), _F32),
        pltpu.SemaphoreType.DMA,
        pltpu.SemaphoreType.DMA,
        pltpu.SemaphoreType.DMA,
        pltpu.SemaphoreType.DMA,
    ],
)
def _hop_kernel(y_hbm, ric_hbm, z128_hbm, t_hbm,
                iva, gb0, gb1, acc, sg0, sg1, ss0, ss1):
    c = lax.axis_index("core")
    s = lax.axis_index("subcore")
    gb = (gb0, gb1)
    sg = (sg0, sg1)
    ss = (ss0, ss1)

    pltpu.sync_copy(z128_hbm, acc.at[pl.ds(s * RPS, RPS)])
    plsc.subcore_barrier()

    def gather(j, b):
        pltpu.async_copy(y_hbm.at[iva.at[j, 0]], gb[b], sg[b])

    def wait_gather(b):
        pltpu.make_async_copy(y_hbm.at[iva.at[0, 0]], gb[b], sg[b]).wait()

    def scatter(j, b):
        pltpu.async_copy(gb[b], acc.at[iva.at[j, 1]], ss[b], add=True)

    def wait_scatter(b):
        pltpu.make_async_copy(gb[b], acc.at[iva.at[0, 1]], ss[b]).wait()

    base_cid = c * NCHC + s * NCH

    @pl.loop(0, NSUP)
    def _(sp):
        pltpu.sync_copy(ric_hbm.at[pl.ds(base_cid + sp * SUPER, SUPER)], iva)
        gather(0, 0)
        gather(1, 1)
        wait_gather(0)
        scatter(0, 0)

        @pl.loop(0, (SUPER - 2) // 2)
        def _(u):
            j0 = 2 * u + 2
            for db in range(2):
                j = j0 + db
                b = db
                wait_scatter(b)
                gather(j, b)
                wait_gather(1 - b)
                scatter(j - 1, 1 - b)

        wait_gather(1)
        scatter(SUPER - 1, 1)
        wait_scatter(0)
        wait_scatter(1)

    plsc.subcore_barrier()
    pltpu.sync_copy(acc.at[pl.ds(s * RPS, RPS)], t_hbm.at[c, pl.ds(s * RPS, RPS)])


_BN = 1000


def _scale1_body(ds_ref, x_ref, yc_ref, dvo_ref, dvi_ref):
    ds = ds_ref[...]
    deg_o = jnp.sum(ds[:, 0], axis=(0, 2))
    deg_i = jnp.sum(ds[:, 1], axis=(0, 2))
    dinv_o = 1.0 / deg_o
    dinv_i = 1.0 / deg_i
    x = x_ref[...]
    yo = x * dinv_o[:, None]
    yi = x * dinv_i[:, None]
    yc_ref[0] = jnp.concatenate([yo[:, :HALF], yi[:, :HALF]], axis=1)
    yc_ref[1] = jnp.concatenate([yo[:, HALF:], yi[:, HALF:]], axis=1)
    dvo_ref[...] = dinv_o[:, None]
    dvi_ref[...] = dinv_i[:, None]


_scale1 = pl.pallas_call(
    _scale1_body,
    grid=(N // _BN,),
    in_specs=[
        pl.BlockSpec((NC, 2, _BN, LANES), lambda i: (0, 0, i, 0)),
        pl.BlockSpec((_BN, D), lambda i: (i, 0)),
    ],
    out_specs=[
        pl.BlockSpec((NC, _BN, D), lambda i: (0, i, 0)),
        pl.BlockSpec((_BN, 1), lambda i: (i, 0)),
        pl.BlockSpec((_BN, 1), lambda i: (i, 0)),
    ],
    out_shape=[
        jax.ShapeDtypeStruct((NC, N, D), _F32),
        jax.ShapeDtypeStruct((N, 1), _F32),
        jax.ShapeDtypeStruct((N, 1), _F32),
    ],
)


def _scale2_body(t_ref, dvo_ref, dvi_ref, yc_ref):
    dinv_o = dvo_ref[...]
    dinv_i = dvi_ref[...]
    for cc in range(NC):
        t = t_ref[cc]
        yc_ref[cc] = jnp.concatenate(
            [t[:, :HALF] * dinv_o, t[:, HALF:] * dinv_i], axis=1)


_scale2 = pl.pallas_call(
    _scale2_body,
    grid=(N // _BN,),
    in_specs=[
        pl.BlockSpec((NC, _BN, D), lambda i: (0, i, 0)),
        pl.BlockSpec((_BN, 1), lambda i: (i, 0)),
        pl.BlockSpec((_BN, 1), lambda i: (i, 0)),
    ],
    out_specs=pl.BlockSpec((NC, _BN, D), lambda i: (0, i, 0)),
    out_shape=jax.ShapeDtypeStruct((NC, N, D), _F32),
)


def _final_body(x_ref, t1_ref, p2_ref, wx_ref, wh_ref, b_ref, out_ref):
    dn = (((1,), (0,)), ((), ()))
    hi = lax.Precision.HIGHEST
    o = lax.dot_general(x_ref[...], wx_ref[...], dn, precision=hi)
    for a, ref in enumerate((t1_ref, p2_ref)):
        for cc in range(NC):
            t = ref[cc]
            o += lax.dot_general(t[:, :HALF], wh_ref[a, cc, 0], dn, precision=hi)
            o += lax.dot_general(t[:, HALF:], wh_ref[a, cc, 1], dn, precision=hi)
    o += b_ref[...]
    z = jax.nn.sigmoid(o[:, :D])
    ht = jnp.tanh(o[:, D:])
    out_ref[...] = (1.0 - z) * ht


_final = pl.pallas_call(
    _final_body,
    grid=(N // _BN,),
    in_specs=[
        pl.BlockSpec((_BN, D), lambda i: (i, 0)),
        pl.BlockSpec((NC, _BN, D), lambda i: (0, i, 0)),
        pl.BlockSpec((NC, _BN, D), lambda i: (0, i, 0)),
        pl.BlockSpec((D, 2 * D), lambda i: (0, 0)),
        pl.BlockSpec((2, NC, 2, HALF, 2 * D), lambda i: (0, 0, 0, 0, 0)),
        pl.BlockSpec((1, 2 * D), lambda i: (0, 0)),
    ],
    out_specs=pl.BlockSpec((_BN, D), lambda i: (i, 0)),
    out_shape=jax.ShapeDtypeStruct((N, D), _F32),
)


def kernel(X, edge_index, edge_weight, W_z, b_z, W_r, b_r, W_h, b_h):
    row = edge_index[0].astype(jnp.int32)
    col = edge_index[1].astype(jnp.int32)
    w = edge_weight.astype(_F32)
    z16 = jnp.zeros((RPS, LANES), _F32)
    z128 = jnp.zeros((RPS, D), _F32)

    rows_c = row.reshape(NCHC, 1, CHUNK)
    cols_c = col.reshape(NCHC, 1, CHUNK)
    ric = jnp.concatenate([
        jnp.concatenate([rows_c, cols_c], axis=1),
        jnp.concatenate([rows_c + N, cols_c], axis=1),
    ], axis=0)

    ds = _deg_kernel(w, row, col, z16).reshape(NC, 2, N_PAD, LANES)
    y1, dvo, dvi = _scale1(ds, X)
    t1 = _hop_kernel(y1.reshape(NC * N, D), ric, z128)
    y2 = _scale2(t1, dvo, dvi)
    p2 = _hop_kernel(y2.reshape(NC * N, D), ric, z128)

    def eff(Wg):
        Wk = Wg[:, :, :D, :]
        wa = Wk[0, 0] + Wk[1, 0] - Wk[0, 2] - Wk[1, 2]
        return wa, Wk[0, 1], Wk[1, 1], 2.0 * Wk[0, 2], 2.0 * Wk[1, 2]

    az = eff(W_z)
    ah = eff(W_h)
    wx = jnp.concatenate([az[0], ah[0]], axis=1)
    wh_rows = []
    for a in range(2):
        core_rows = []
        for cc in range(NC):
            kind_rows = []
            for k in range(2):
                wcat = jnp.concatenate(
                    [az[1 + 2 * a + k], ah[1 + 2 * a + k]], axis=1)
                kind_rows.append(wcat[cc * HALF:(cc + 1) * HALF])
            core_rows.append(jnp.stack(kind_rows))
        wh_rows.append(jnp.stack(core_rows))
    wh = jnp.stack(wh_rows)
    b2 = jnp.concatenate([b_z, b_h]).reshape(1, 2 * D)

    return _final(X, t1, p2, wx, wh, b2)


# speedup vs baseline: 22.2471x; 1.1116x over previous
import functools

import jax
import jax.numpy as jnp
from jax import lax
from jax.experimental import pallas as pl
from jax.experimental.pallas import tpu as pltpu
from jax.experimental.pallas import tpu_sc as plsc

N = 10000
D = 128
HALF = 64
E = 320000
NC = 2
NS = 16
LANES = 16
N_PAD = 10240
RPS = N_PAD // NS
CHUNK = 80

_MESH = plsc.VectorSubcoreMesh(core_axis_name="core", subcore_axis_name="subcore")

_F32 = jnp.float32


@functools.partial(
    pl.kernel,
    out_type=jax.ShapeDtypeStruct((NC * 2 * N_PAD, LANES), _F32),
    mesh=_MESH,
    scratch_types=[
        pltpu.VMEM((3, CHUNK), jnp.int32),
        pltpu.VMEM((CHUNK, LANES), _F32),
        pltpu.VMEM_SHARED((N_PAD, LANES), _F32),
        pltpu.VMEM_SHARED((N_PAD, LANES), _F32),
    ],
)
def _deg_kernel(rcw_hbm, z16_hbm, out_hbm, iv3, mv, acc_o, acc_i):
    c = lax.axis_index("core")
    s = lax.axis_index("subcore")
    wid = c * NS + s
    iota = lax.broadcasted_iota(jnp.int32, (LANES,), 0)

    pltpu.sync_copy(z16_hbm, acc_o.at[pl.ds(s * RPS, RPS)])
    pltpu.sync_copy(z16_hbm, acc_i.at[pl.ds(s * RPS, RPS)])
    plsc.subcore_barrier()

    nchw = E // (NC * NS) // CHUNK
    base_c = wid * nchw

    @pl.loop(0, nchw)
    def _(i):
        pltpu.sync_copy(rcw_hbm.at[base_c + i], iv3)
        for g in range(CHUNK // LANES):
            w16 = lax.bitcast_convert_type(iv3[2, pl.ds(g * LANES, LANES)], _F32)
            for b in range(LANES):
                mv[g * LANES + b, :] = jnp.where(iota == b, w16, 0.0)
        pltpu.sync_copy(mv, acc_o.at[iv3.at[0]], add=True)
        pltpu.sync_copy(mv, acc_i.at[iv3.at[1]], add=True)

    plsc.subcore_barrier()
    pltpu.sync_copy(acc_o.at[pl.ds(s * RPS, RPS)],
                    out_hbm.at[pl.ds((c * 2) * N_PAD + s * RPS, RPS)])
    pltpu.sync_copy(acc_i.at[pl.ds(s * RPS, RPS)],
                    out_hbm.at[pl.ds((c * 2 + 1) * N_PAD + s * RPS, RPS)])


NCH = E // NS // CHUNK
NCHC = E // CHUNK
SUPER = 50
NSUP = NCH // SUPER


@functools.partial(
    pl.kernel,
    out_type=jax.ShapeDtypeStruct((NC, N_PAD, D), _F32),
    mesh=_MESH,
    scratch_types=[
        pltpu.VMEM((SUPER, 2, CHUNK), jnp.int32),
        pltpu.VMEM((CHUNK, D), _F32),
        pltpu.VMEM((CHUNK, D), _F32),
        pltpu.VMEM_SHARED((N_PAD, D), _F32),
        pltpu.SemaphoreType.DMA,
        pltpu.SemaphoreType.DMA,
        pltpu.SemaphoreType.DMA,
        pltpu.SemaphoreType.DMA,
    ],
)
def _hop_kernel(y_hbm, ric_hbm, z128_hbm, t_hbm,
                iva, gb0, gb1, acc, sg0, sg1, ss0, ss1):
    c = lax.axis_index("core")
    s = lax.axis_index("subcore")
    gb = (gb0, gb1)
    sg = (sg0, sg1)
    ss = (ss0, ss1)

    pltpu.sync_copy(z128_hbm, acc.at[pl.ds(s * RPS, RPS)])
    plsc.subcore_barrier()

    def gather(j, b):
        pltpu.async_copy(y_hbm.at[iva.at[j, 0]], gb[b], sg[b])

    def wait_gather(b):
        pltpu.make_async_copy(y_hbm.at[iva.at[0, 0]], gb[b], sg[b]).wait()

    def scatter(j, b):
        pltpu.async_copy(gb[b], acc.at[iva.at[j, 1]], ss[b], add=True)

    def wait_scatter(b):
        pltpu.make_async_copy(gb[b], acc.at[iva.at[0, 1]], ss[b]).wait()

    base_cid = c * NCHC + s * NCH

    @pl.loop(0, NSUP)
    def _(sp):
        pltpu.sync_copy(ric_hbm.at[pl.ds(base_cid + sp * SUPER, SUPER)], iva)
        gather(0, 0)
        gather(1, 1)
        wait_gather(0)
        scatter(0, 0)

        @pl.loop(0, (SUPER - 2) // 2)
        def _(u):
            j0 = 2 * u + 2
            for db in range(2):
                j = j0 + db
                b = db
                wait_scatter(b)
                gather(j, b)
                wait_gather(1 - b)
                scatter(j - 1, 1 - b)

        wait_gather(1)
        scatter(SUPER - 1, 1)
        wait_scatter(0)
        wait_scatter(1)

    plsc.subcore_barrier()
    pltpu.sync_copy(acc.at[pl.ds(s * RPS, RPS)], t_hbm.at[c, pl.ds(s * RPS, RPS)])


_BN = 1000


def _scale1_body(ds_ref, x_ref, yc_ref, dvo_ref, dvi_ref):
    ds = ds_ref[...]
    deg_o = jnp.sum(ds[:, 0], axis=(0, 2))
    deg_i = jnp.sum(ds[:, 1], axis=(0, 2))
    dinv_o = 1.0 / deg_o
    dinv_i = 1.0 / deg_i
    x = x_ref[...]
    yo = x * dinv_o[:, None]
    yi = x * dinv_i[:, None]
    yc_ref[0] = jnp.concatenate([yo[:, :HALF], yi[:, :HALF]], axis=1)
    yc_ref[1] = jnp.concatenate([yo[:, HALF:], yi[:, HALF:]], axis=1)
    dvo_ref[...] = dinv_o[:, None]
    dvi_ref[...] = dinv_i[:, None]


_scale1 = pl.pallas_call(
    _scale1_body,
    grid=(N // _BN,),
    in_specs=[
        pl.BlockSpec((NC, 2, _BN, LANES), lambda i: (0, 0, i, 0)),
        pl.BlockSpec((_BN, D), lambda i: (i, 0)),
    ],
    out_specs=[
        pl.BlockSpec((NC, _BN, D), lambda i: (0, i, 0)),
        pl.BlockSpec((_BN, 1), lambda i: (i, 0)),
        pl.BlockSpec((_BN, 1), lambda i: (i, 0)),
    ],
    out_shape=[
        jax.ShapeDtypeStruct((NC, N, D), _F32),
        jax.ShapeDtypeStruct((N, 1), _F32),
        jax.ShapeDtypeStruct((N, 1), _F32),
    ],
)


def _scale2_body(t_ref, dvo_ref, dvi_ref, yc_ref):
    dinv_o = dvo_ref[...]
    dinv_i = dvi_ref[...]
    for cc in range(NC):
        t = t_ref[cc]
        yc_ref[cc] = jnp.concatenate(
            [t[:, :HALF] * dinv_o, t[:, HALF:] * dinv_i], axis=1)


_scale2 = pl.pallas_call(
    _scale2_body,
    grid=(N // _BN,),
    in_specs=[
        pl.BlockSpec((NC, _BN, D), lambda i: (0, i, 0)),
        pl.BlockSpec((_BN, 1), lambda i: (i, 0)),
        pl.BlockSpec((_BN, 1), lambda i: (i, 0)),
    ],
    out_specs=pl.BlockSpec((NC, _BN, D), lambda i: (0, i, 0)),
    out_shape=jax.ShapeDtypeStruct((NC, N, D), _F32),
)


def _final_body(x_ref, t1_ref, p2_ref, wx_ref, wh_ref, b_ref, out_ref):
    dn = (((1,), (0,)), ((), ()))
    hi = lax.Precision.HIGHEST
    o = lax.dot_general(x_ref[...], wx_ref[...], dn, precision=hi)
    for a, ref in enumerate((t1_ref, p2_ref)):
        for cc in range(NC):
            t = ref[cc]
            o += lax.dot_general(t[:, :HALF], wh_ref[a, cc, 0], dn, precision=hi)
            o += lax.dot_general(t[:, HALF:], wh_ref[a, cc, 1], dn, precision=hi)
    o += b_ref[...]
    z = jax.nn.sigmoid(o[:, :D])
    ht = jnp.tanh(o[:, D:])
    out_ref[...] = (1.0 - z) * ht


_final = pl.pallas_call(
    _final_body,
    grid=(N // _BN,),
    in_specs=[
        pl.BlockSpec((_BN, D), lambda i: (i, 0)),
        pl.BlockSpec((NC, _BN, D), lambda i: (0, i, 0)),
        pl.BlockSpec((NC, _BN, D), lambda i: (0, i, 0)),
        pl.BlockSpec((D, 2 * D), lambda i: (0, 0)),
        pl.BlockSpec((2, NC, 2, HALF, 2 * D), lambda i: (0, 0, 0, 0, 0)),
        pl.BlockSpec((1, 2 * D), lambda i: (0, 0)),
    ],
    out_specs=pl.BlockSpec((_BN, D), lambda i: (i, 0)),
    out_shape=jax.ShapeDtypeStruct((N, D), _F32),
)


def kernel(X, edge_index, edge_weight, W_z, b_z, W_r, b_r, W_h, b_h):
    row = edge_index[0].astype(jnp.int32)
    col = edge_index[1].astype(jnp.int32)
    w = edge_weight.astype(_F32)
    z16 = jnp.zeros((RPS, LANES), _F32)
    z128 = jnp.zeros((RPS, D), _F32)

    rows_c = row.reshape(NCHC, 1, CHUNK)
    cols_c = col.reshape(NCHC, 1, CHUNK)
    ric = jnp.concatenate([
        jnp.concatenate([rows_c, cols_c], axis=1),
        jnp.concatenate([rows_c + N, cols_c], axis=1),
    ], axis=0)

    wbits_c = lax.bitcast_convert_type(w, jnp.int32).reshape(NCHC, 1, CHUNK)
    rcw = jnp.concatenate([rows_c, cols_c, wbits_c], axis=1)
    ds = _deg_kernel(rcw, z16).reshape(NC, 2, N_PAD, LANES)
    y1, dvo, dvi = _scale1(ds, X)
    t1 = _hop_kernel(y1.reshape(NC * N, D), ric, z128)
    y2 = _scale2(t1, dvo, dvi)
    p2 = _hop_kernel(y2.reshape(NC * N, D), ric, z128)

    def eff(Wg):
        Wk = Wg[:, :, :D, :]
        wa = Wk[0, 0] + Wk[1, 0] - Wk[0, 2] - Wk[1, 2]
        return wa, Wk[0, 1], Wk[1, 1], 2.0 * Wk[0, 2], 2.0 * Wk[1, 2]

    az = eff(W_z)
    ah = eff(W_h)
    wx = jnp.concatenate([az[0], ah[0]], axis=1)
    wh_rows = []
    for a in range(2):
        core_rows = []
        for cc in range(NC):
            kind_rows = []
            for k in range(2):
                wcat = jnp.concatenate(
                    [az[1 + 2 * a + k], ah[1 + 2 * a + k]], axis=1)
                kind_rows.append(wcat[cc * HALF:(cc + 1) * HALF])
            core_rows.append(jnp.stack(kind_rows))
        wh_rows.append(jnp.stack(core_rows))
    wh = jnp.stack(wh_rows)
    b2 = jnp.concatenate([b_z, b_h]).reshape(1, 2 * D)

    return _final(X, t1, p2, wx, wh, b2)


# speedup vs baseline: 24.0846x; 1.0826x over previous
import functools

import jax
import jax.numpy as jnp
from jax import lax
from jax.experimental import pallas as pl
from jax.experimental.pallas import tpu as pltpu
from jax.experimental.pallas import tpu_sc as plsc

N = 10000
D = 128
HALF = 64
E = 320000
NC = 2
NS = 16
LANES = 16
N_PAD = 10240
RPS = N_PAD // NS
CHUNK = 80

_MESH = plsc.VectorSubcoreMesh(core_axis_name="core", subcore_axis_name="subcore")

_F32 = jnp.float32


@functools.partial(
    pl.kernel,
    out_type=jax.ShapeDtypeStruct((NC * 2 * N_PAD, LANES), _F32),
    mesh=_MESH,
    scratch_types=[
        pltpu.VMEM((3, CHUNK), jnp.int32),
        pltpu.VMEM((CHUNK, LANES), _F32),
        pltpu.VMEM_SHARED((N_PAD, LANES), _F32),
        pltpu.VMEM_SHARED((N_PAD, LANES), _F32),
    ],
)
def _deg_kernel(rcw_hbm, z16_hbm, out_hbm, iv3, mv, acc_o, acc_i):
    c = lax.axis_index("core")
    s = lax.axis_index("subcore")
    wid = c * NS + s
    iota = lax.broadcasted_iota(jnp.int32, (LANES,), 0)

    pltpu.sync_copy(z16_hbm, acc_o.at[pl.ds(s * RPS, RPS)])
    pltpu.sync_copy(z16_hbm, acc_i.at[pl.ds(s * RPS, RPS)])
    plsc.subcore_barrier()

    nchw = E // (NC * NS) // CHUNK
    base_c = wid * nchw

    @pl.loop(0, nchw)
    def _(i):
        pltpu.sync_copy(rcw_hbm.at[base_c + i], iv3)
        for g in range(CHUNK // LANES):
            w16 = lax.bitcast_convert_type(iv3[2, pl.ds(g * LANES, LANES)], _F32)
            for b in range(LANES):
                mv[g * LANES + b, :] = jnp.where(iota == b, w16, 0.0)
        pltpu.sync_copy(mv, acc_o.at[iv3.at[0]], add=True)
        pltpu.sync_copy(mv, acc_i.at[iv3.at[1]], add=True)

    plsc.subcore_barrier()
    pltpu.sync_copy(acc_o.at[pl.ds(s * RPS, RPS)],
                    out_hbm.at[pl.ds((c * 2) * N_PAD + s * RPS, RPS)])
    pltpu.sync_copy(acc_i.at[pl.ds(s * RPS, RPS)],
                    out_hbm.at[pl.ds((c * 2 + 1) * N_PAD + s * RPS, RPS)])


NCH = E // NS // CHUNK
NCHC = E // CHUNK
SUPER = 50
NSUP = NCH // SUPER


@functools.partial(
    pl.kernel,
    out_type=jax.ShapeDtypeStruct((NC, N_PAD, D), _F32),
    mesh=_MESH,
    scratch_types=[
        pltpu.VMEM((SUPER, 2, CHUNK), jnp.int32),
        pltpu.VMEM((CHUNK, D), _F32),
        pltpu.VMEM((CHUNK, D), _F32),
        pltpu.VMEM_SHARED((N_PAD, D), _F32),
        pltpu.SemaphoreType.DMA,
        pltpu.SemaphoreType.DMA,
        pltpu.SemaphoreType.DMA,
        pltpu.SemaphoreType.DMA,
    ],
)
def _hop_kernel(y_hbm, ric_hbm, z128_hbm, t_hbm,
                iva, gb0, gb1, acc, sg0, sg1, ss0, ss1):
    c = lax.axis_index("core")
    s = lax.axis_index("subcore")
    gb = (gb0, gb1)
    sg = (sg0, sg1)
    ss = (ss0, ss1)

    pltpu.sync_copy(z128_hbm, acc.at[pl.ds(s * RPS, RPS)])
    plsc.subcore_barrier()

    def gather(j, b):
        pltpu.async_copy(y_hbm.at[iva.at[j, 0]], gb[b], sg[b])

    def wait_gather(b):
        pltpu.make_async_copy(y_hbm.at[iva.at[0, 0]], gb[b], sg[b]).wait()

    def scatter(j, b):
        pltpu.async_copy(gb[b], acc.at[iva.at[j, 1]], ss[b], add=True)

    def wait_scatter(b):
        pltpu.make_async_copy(gb[b], acc.at[iva.at[0, 1]], ss[b]).wait()

    base_cid = c * NCHC + s * NCH

    @pl.loop(0, NSUP)
    def _(sp):
        pltpu.sync_copy(ric_hbm.at[pl.ds(base_cid + sp * SUPER, SUPER)], iva)
        gather(0, 0)
        gather(1, 1)
        wait_gather(0)
        scatter(0, 0)

        @pl.loop(0, (SUPER - 2) // 2)
        def _(u):
            j0 = 2 * u + 2
            for db in range(2):
                j = j0 + db
                b = db
                wait_scatter(b)
                gather(j, b)
                wait_gather(1 - b)
                scatter(j - 1, 1 - b)

        wait_gather(1)
        scatter(SUPER - 1, 1)
        wait_scatter(0)
        wait_scatter(1)

    plsc.subcore_barrier()
    pltpu.sync_copy(acc.at[pl.ds(s * RPS, RPS)], t_hbm.at[c, pl.ds(s * RPS, RPS)])


_BN = 1000


def _scale1_body(ds_ref, x_ref, yc_ref, dvo_ref, dvi_ref):
    ds = ds_ref[...]
    deg_o = jnp.sum(ds[:, 0], axis=(0, 2))
    deg_i = jnp.sum(ds[:, 1], axis=(0, 2))
    dinv_o = 1.0 / deg_o
    dinv_i = 1.0 / deg_i
    x = x_ref[...]
    yo = x * dinv_o[:, None]
    yi = x * dinv_i[:, None]
    yc_ref[0] = jnp.concatenate([yo[:, :HALF], yi[:, :HALF]], axis=1)
    yc_ref[1] = jnp.concatenate([yo[:, HALF:], yi[:, HALF:]], axis=1)
    dvo_ref[...] = dinv_o[:, None]
    dvi_ref[...] = dinv_i[:, None]


_scale1 = pl.pallas_call(
    _scale1_body,
    grid=(N // _BN,),
    in_specs=[
        pl.BlockSpec((NC, 2, _BN, LANES), lambda i: (0, 0, i, 0)),
        pl.BlockSpec((_BN, D), lambda i: (i, 0)),
    ],
    out_specs=[
        pl.BlockSpec((NC, _BN, D), lambda i: (0, i, 0)),
        pl.BlockSpec((_BN, 1), lambda i: (i, 0)),
        pl.BlockSpec((_BN, 1), lambda i: (i, 0)),
    ],
    out_shape=[
        jax.ShapeDtypeStruct((NC, N, D), _F32),
        jax.ShapeDtypeStruct((N, 1), _F32),
        jax.ShapeDtypeStruct((N, 1), _F32),
    ],
)


def _scale2_body(t_ref, dvo_ref, dvi_ref, yc_ref):
    dinv_o = dvo_ref[...]
    dinv_i = dvi_ref[...]
    for cc in range(NC):
        t = t_ref[cc]
        yc_ref[cc] = jnp.concatenate(
            [t[:, :HALF] * dinv_o, t[:, HALF:] * dinv_i], axis=1)


_scale2 = pl.pallas_call(
    _scale2_body,
    grid=(N // _BN,),
    in_specs=[
        pl.BlockSpec((NC, _BN, D), lambda i: (0, i, 0)),
        pl.BlockSpec((_BN, 1), lambda i: (i, 0)),
        pl.BlockSpec((_BN, 1), lambda i: (i, 0)),
    ],
    out_specs=pl.BlockSpec((NC, _BN, D), lambda i: (0, i, 0)),
    out_shape=jax.ShapeDtypeStruct((NC, N, D), _F32),
)


def _final_body(x_ref, t1_ref, p2_ref, wx_ref, wh_ref, b_ref, out_ref):
    dn = (((1,), (0,)), ((), ()))
    hi = lax.Precision.DEFAULT
    o = lax.dot_general(x_ref[...], wx_ref[...], dn, precision=hi)
    for a, ref in enumerate((t1_ref, p2_ref)):
        for cc in range(NC):
            t = ref[cc]
            o += lax.dot_general(t[:, :HALF], wh_ref[a, cc, 0], dn, precision=hi)
            o += lax.dot_general(t[:, HALF:], wh_ref[a, cc, 1], dn, precision=hi)
    o += b_ref[...]
    z = jax.nn.sigmoid(o[:, :D])
    ht = jnp.tanh(o[:, D:])
    out_ref[...] = (1.0 - z) * ht


_final = pl.pallas_call(
    _final_body,
    grid=(N // _BN,),
    in_specs=[
        pl.BlockSpec((_BN, D), lambda i: (i, 0)),
        pl.BlockSpec((NC, _BN, D), lambda i: (0, i, 0)),
        pl.BlockSpec((NC, _BN, D), lambda i: (0, i, 0)),
        pl.BlockSpec((D, 2 * D), lambda i: (0, 0)),
        pl.BlockSpec((2, NC, 2, HALF, 2 * D), lambda i: (0, 0, 0, 0, 0)),
        pl.BlockSpec((1, 2 * D), lambda i: (0, 0)),
    ],
    out_specs=pl.BlockSpec((_BN, D), lambda i: (i, 0)),
    out_shape=jax.ShapeDtypeStruct((N, D), _F32),
)


def kernel(X, edge_index, edge_weight, W_z, b_z, W_r, b_r, W_h, b_h):
    row = edge_index[0].astype(jnp.int32)
    col = edge_index[1].astype(jnp.int32)
    w = edge_weight.astype(_F32)
    z16 = jnp.zeros((RPS, LANES), _F32)
    z128 = jnp.zeros((RPS, D), _F32)

    rows_c = row.reshape(NCHC, 1, CHUNK)
    cols_c = col.reshape(NCHC, 1, CHUNK)
    ric = jnp.concatenate([
        jnp.concatenate([rows_c, cols_c], axis=1),
        jnp.concatenate([rows_c + N, cols_c], axis=1),
    ], axis=0)

    wbits_c = lax.bitcast_convert_type(w, jnp.int32).reshape(NCHC, 1, CHUNK)
    rcw = jnp.concatenate([rows_c, cols_c, wbits_c], axis=1)
    ds = _deg_kernel(rcw, z16).reshape(NC, 2, N_PAD, LANES)
    y1, dvo, dvi = _scale1(ds, X)
    t1 = _hop_kernel(y1.reshape(NC * N, D), ric, z128)
    y2 = _scale2(t1, dvo, dvi)
    p2 = _hop_kernel(y2.reshape(NC * N, D), ric, z128)

    def eff(Wg):
        Wk = Wg[:, :, :D, :]
        wa = Wk[0, 0] + Wk[1, 0] - Wk[0, 2] - Wk[1, 2]
        return wa, Wk[0, 1], Wk[1, 1], 2.0 * Wk[0, 2], 2.0 * Wk[1, 2]

    az = eff(W_z)
    ah = eff(W_h)
    wx = jnp.concatenate([az[0], ah[0]], axis=1)
    wh_rows = []
    for a in range(2):
        core_rows = []
        for cc in range(NC):
            kind_rows = []
            for k in range(2):
                wcat = jnp.concatenate(
                    [az[1 + 2 * a + k], ah[1 + 2 * a + k]], axis=1)
                kind_rows.append(wcat[cc * HALF:(cc + 1) * HALF])
            core_rows.append(jnp.stack(kind_rows))
        wh_rows.append(jnp.stack(core_rows))
    wh = jnp.stack(wh_rows)
    b2 = jnp.concatenate([b_z, b_h]).reshape(1, 2 * D)

    return _final(X, t1, p2, wx, wh, b2)
